# baseline (device time: 17460 ns/iter reference)
import numpy as np

import jax
import jax.numpy as jnp
from jax import lax
from jax.experimental import pallas as pl
from jax.experimental.pallas import tpu as pltpu

N_DEV = 8
B, SQ, D = 2, 128, 512
DH = 64
R = B * SQ
CH2 = R // N_DEV


def _rope_tables(HD):
    lane = np.arange(HD)
    inv = 10000.0 ** (-(2.0 * ((lane % DH) // 2)) / DH)
    ang = np.arange(SQ)[:, None] * inv[None, :]
    cos = np.cos(ang).astype(np.float32)
    sin = np.sin(ang).astype(np.float32)
    return cos, sin


def kernel(x, Wq, Wk, Wv, Wo):
    HL = Wq.shape[1] // DH
    HD = HL * DH
    cos_np, sin_np = _rope_tables(HD)

    def body(a_ref, b_ref, out_ref,
             pbuf_ref, rbuf1_ref, obuf_ref,
             s1_sems, r1_sems, s2_sems, r2_sems, o_sem):
        my = lax.axis_index("i")

        barrier = pltpu.get_barrier_semaphore()
        for t in range(N_DEV):
            @pl.when(t != my)
            def _(t=t):
                pl.semaphore_signal(barrier, inc=1, device_id=(t,),
                                    device_id_type=pl.DeviceIdType.MESH)

        xb = a_ref[:R, :].astype(jnp.bfloat16)
        wo = a_ref[R:, :].astype(jnp.bfloat16)
        wq = b_ref[0 * D:1 * D, :].astype(jnp.bfloat16)
        wk = b_ref[1 * D:2 * D, :].astype(jnp.bfloat16)
        wv = b_ref[2 * D:3 * D, :].astype(jnp.bfloat16)
        c1 = b_ref[3 * D:3 * D + SQ, :]
        s1 = b_ref[3 * D + SQ:3 * D + 2 * SQ, :]

        q0 = jnp.dot(xb, wq, preferred_element_type=jnp.float32)
        k0 = jnp.dot(xb, wk, preferred_element_type=jnp.float32)
        v = jnp.dot(xb, wv, preferred_element_type=jnp.float32
                    ).astype(jnp.bfloat16)
        qk = jnp.concatenate([q0, k0], axis=0)

        ii = lax.broadcasted_iota(jnp.int32, (HD, HD), 0)
        jj = lax.broadcasted_iota(jnp.int32, (HD, HD), 1)
        even_j = (jj % 2) == 0
        perm = jnp.where((ii == jj + 1) & even_j, -1.0,
                         jnp.where((jj == ii + 1) & ~even_j, 1.0, 0.0)
                         ).astype(jnp.bfloat16)
        qk_r = jnp.dot(qk.astype(jnp.bfloat16), perm,
                       preferred_element_type=jnp.float32)
        cos4 = jnp.concatenate([c1, c1, c1, c1], axis=0)
        sin4 = jnp.concatenate([s1, s1, s1, s1], axis=0)
        qk16 = (qk * cos4 + qk_r * sin4).astype(jnp.bfloat16)
        q = qk16[:R, :]
        k = qk16[R:, :]

        ctx_rows = []
        for b in range(B):
            r = slice(b * SQ, (b + 1) * SQ)
            ctxs = []
            for h in range(HL):
                c = slice(h * DH, (h + 1) * DH)
                s = jnp.dot(q[r, c], k[r, c].T,
                            preferred_element_type=jnp.float32) * 0.125
                w = jnp.exp(s)
                w = (w / jnp.sum(w, axis=-1, keepdims=True)
                     ).astype(jnp.bfloat16)
                ctxs.append(jnp.dot(w, v[r, c],
                                    preferred_element_type=jnp.float32))
            ctx_rows.append(jnp.concatenate(ctxs, axis=1))
        ctx = jnp.concatenate(ctx_rows, axis=0)
        pb = jnp.dot(ctx.astype(jnp.bfloat16), wo,
                     preferred_element_type=jnp.float32)
        pbuf_ref[:, :] = pb.astype(jnp.bfloat16)
        pl.semaphore_wait(barrier, N_DEV - 1)

        for j in range(N_DEV):
            @pl.when(j != my)
            def _(j=j):
                rdma = pltpu.make_async_remote_copy(
                    src_ref=pbuf_ref.at[pl.ds(j * CH2, CH2), :],
                    dst_ref=rbuf1_ref.at[my],
                    send_sem=s1_sems.at[j],
                    recv_sem=r1_sems.at[my],
                    device_id=(j,),
                    device_id_type=pl.DeviceIdType.MESH,
                )
                rdma.start()
        rbuf1_ref[my] = pbuf_ref[pl.ds(my * CH2, CH2), :]
        for s in range(N_DEV):
            @pl.when(s != my)
            def _(s=s):
                recv = pltpu.make_async_remote_copy(
                    src_ref=rbuf1_ref.at[s],
                    dst_ref=rbuf1_ref.at[s],
                    send_sem=s1_sems.at[s],
                    recv_sem=r1_sems.at[s],
                    device_id=(0,),
                    device_id_type=pl.DeviceIdType.MESH,
                )
                recv.wait_recv()
        acc = rbuf1_ref[0, :, :].astype(jnp.float32)
        for s in range(1, N_DEV):
            acc = acc + rbuf1_ref[s, :, :].astype(jnp.float32)
        obuf_ref[:, :] = acc

        dst = out_ref.at[my // 4, pl.ds((my % 4) * CH2, CH2), :]
        for t in range(N_DEV):
            @pl.when(t != my)
            def _(t=t):
                rdma = pltpu.make_async_remote_copy(
                    src_ref=obuf_ref,
                    dst_ref=dst,
                    send_sem=s2_sems.at[t],
                    recv_sem=r2_sems.at[my],
                    device_id=(t,),
                    device_id_type=pl.DeviceIdType.MESH,
                )
                rdma.start()
        own = pltpu.make_async_copy(obuf_ref, dst, o_sem)
        own.start()
        for s in range(N_DEV):
            @pl.when(s != my)
            def _(s=s):
                recv = pltpu.make_async_remote_copy(
                    src_ref=obuf_ref,
                    dst_ref=dst,
                    send_sem=s2_sems.at[s],
                    recv_sem=r2_sems.at[s],
                    device_id=(0,),
                    device_id_type=pl.DeviceIdType.MESH,
                )
                recv.wait_recv()
        own.wait()

        for t in range(N_DEV):
            @pl.when(t != my)
            def _(t=t):
                w1 = pltpu.make_async_remote_copy(
                    src_ref=rbuf1_ref.at[t], dst_ref=rbuf1_ref.at[t],
                    send_sem=s1_sems.at[t], recv_sem=r1_sems.at[t],
                    device_id=(0,), device_id_type=pl.DeviceIdType.MESH,
                )
                w1.wait_send()
                w2 = pltpu.make_async_remote_copy(
                    src_ref=obuf_ref, dst_ref=obuf_ref,
                    send_sem=s2_sems.at[t], recv_sem=r2_sems.at[t],
                    device_id=(0,), device_id_type=pl.DeviceIdType.MESH,
                )
                w2.wait_send()

    A = jnp.concatenate([x.reshape(R, D), Wo], axis=0)
    Bm = jnp.concatenate([Wq, Wk, Wv,
                          jnp.asarray(cos_np), jnp.asarray(sin_np)],
                         axis=0)

    return pl.pallas_call(
        body,
        out_shape=jax.ShapeDtypeStruct((B, SQ, D), jnp.float32),
        in_specs=[pl.BlockSpec(memory_space=pltpu.VMEM)] * 2,
        out_specs=pl.BlockSpec(memory_space=pltpu.MemorySpace.HBM),
        scratch_shapes=[
            pltpu.VMEM((R, D), jnp.bfloat16),
            pltpu.VMEM((N_DEV, CH2, D), jnp.bfloat16),
            pltpu.VMEM((CH2, D), jnp.float32),
            pltpu.SemaphoreType.DMA((N_DEV,)),
            pltpu.SemaphoreType.DMA((N_DEV,)),
            pltpu.SemaphoreType.DMA((N_DEV,)),
            pltpu.SemaphoreType.DMA((N_DEV,)),
            pltpu.SemaphoreType.DMA,
        ],
        compiler_params=pltpu.CompilerParams(collective_id=0),
    )(A, Bm)


# device time: 16002 ns/iter; 1.0911x vs baseline; 1.0911x over previous
import numpy as np

import jax
import jax.numpy as jnp
from jax import lax
from jax.experimental import pallas as pl
from jax.experimental.pallas import tpu as pltpu

N_DEV = 8
B, SQ, D = 2, 128, 512
DH = 64
R = B * SQ
CH2 = R // N_DEV


def _rope_tables(HD):
    lane = np.arange(HD)
    inv = 10000.0 ** (-(2.0 * ((lane % DH) // 2)) / DH)
    ang = np.arange(SQ)[:, None] * inv[None, :]
    cos = np.cos(ang).astype(np.float32)
    sin = np.sin(ang).astype(np.float32)
    return cos, sin


def kernel(x, Wq, Wk, Wv, Wo):
    HL = Wq.shape[1] // DH
    HD = HL * DH
    cos_np, sin_np = _rope_tables(HD)

    def body(a_ref, b_ref, out_ref,
             pbuf_ref, rbuf1_ref, rbuf2_ref,
             s1_sems, r1_sems, s2_sems, r2_sems):
        my = lax.axis_index("i")

        barrier = pltpu.get_barrier_semaphore()
        for t in range(N_DEV):
            @pl.when(t != my)
            def _(t=t):
                pl.semaphore_signal(barrier, inc=1, device_id=(t,),
                                    device_id_type=pl.DeviceIdType.MESH)

        xb = a_ref[:R, :].astype(jnp.bfloat16)
        wo = a_ref[R:, :].astype(jnp.bfloat16)
        wq = b_ref[0 * D:1 * D, :].astype(jnp.bfloat16)
        wk = b_ref[1 * D:2 * D, :].astype(jnp.bfloat16)
        wv = b_ref[2 * D:3 * D, :].astype(jnp.bfloat16)
        c1 = b_ref[3 * D:3 * D + SQ, :]
        s1 = b_ref[3 * D + SQ:3 * D + 2 * SQ, :]

        q0 = jnp.dot(xb, wq, preferred_element_type=jnp.float32)
        k0 = jnp.dot(xb, wk, preferred_element_type=jnp.float32)
        v = jnp.dot(xb, wv, preferred_element_type=jnp.float32
                    ).astype(jnp.bfloat16)
        qk = jnp.concatenate([q0, k0], axis=0)

        ii = lax.broadcasted_iota(jnp.int32, (HD, HD), 0)
        jj = lax.broadcasted_iota(jnp.int32, (HD, HD), 1)
        even_j = (jj % 2) == 0
        perm = jnp.where((ii == jj + 1) & even_j, -1.0,
                         jnp.where((jj == ii + 1) & ~even_j, 1.0, 0.0)
                         ).astype(jnp.bfloat16)
        qk_r = jnp.dot(qk.astype(jnp.bfloat16), perm,
                       preferred_element_type=jnp.float32)
        cos4 = jnp.concatenate([c1, c1, c1, c1], axis=0)
        sin4 = jnp.concatenate([s1, s1, s1, s1], axis=0)
        qk16 = (qk * cos4 + qk_r * sin4).astype(jnp.bfloat16)
        q = qk16[:R, :]
        k = qk16[R:, :]

        ctx_rows = []
        for b in range(B):
            r = slice(b * SQ, (b + 1) * SQ)
            ctxs = []
            for h in range(HL):
                c = slice(h * DH, (h + 1) * DH)
                s = jnp.dot(q[r, c], k[r, c].T,
                            preferred_element_type=jnp.float32) * 0.125
                w = jnp.exp(s)
                w = (w / jnp.sum(w, axis=-1, keepdims=True)
                     ).astype(jnp.bfloat16)
                ctxs.append(jnp.dot(w, v[r, c],
                                    preferred_element_type=jnp.float32))
            ctx_rows.append(jnp.concatenate(ctxs, axis=1))
        ctx = jnp.concatenate(ctx_rows, axis=0)
        pb = jnp.dot(ctx.astype(jnp.bfloat16), wo,
                     preferred_element_type=jnp.float32)
        pbuf_ref[:, :] = pb.astype(jnp.bfloat16)
        pl.semaphore_wait(barrier, N_DEV - 1)

        for j in range(N_DEV):
            @pl.when(j != my)
            def _(j=j):
                rdma = pltpu.make_async_remote_copy(
                    src_ref=pbuf_ref.at[pl.ds(j * CH2, CH2), :],
                    dst_ref=rbuf1_ref.at[my],
                    send_sem=s1_sems.at[j],
                    recv_sem=r1_sems.at[my],
                    device_id=(j,),
                    device_id_type=pl.DeviceIdType.MESH,
                )
                rdma.start()
        rbuf1_ref[my] = pbuf_ref[pl.ds(my * CH2, CH2), :]
        for s in range(N_DEV):
            @pl.when(s != my)
            def _(s=s):
                recv = pltpu.make_async_remote_copy(
                    src_ref=rbuf1_ref.at[s],
                    dst_ref=rbuf1_ref.at[s],
                    send_sem=s1_sems.at[s],
                    recv_sem=r1_sems.at[s],
                    device_id=(0,),
                    device_id_type=pl.DeviceIdType.MESH,
                )
                recv.wait_recv()
        acc = rbuf1_ref[0, :, :].astype(jnp.float32)
        for s in range(1, N_DEV):
            acc = acc + rbuf1_ref[s, :, :].astype(jnp.float32)
        rbuf2_ref[my] = acc.astype(jnp.bfloat16)

        for t in range(N_DEV):
            @pl.when(t != my)
            def _(t=t):
                rdma = pltpu.make_async_remote_copy(
                    src_ref=rbuf2_ref.at[my],
                    dst_ref=rbuf2_ref.at[my],
                    send_sem=s2_sems.at[t],
                    recv_sem=r2_sems.at[my],
                    device_id=(t,),
                    device_id_type=pl.DeviceIdType.MESH,
                )
                rdma.start()
        for s in range(N_DEV):
            @pl.when(s != my)
            def _(s=s):
                recv = pltpu.make_async_remote_copy(
                    src_ref=rbuf2_ref.at[s],
                    dst_ref=rbuf2_ref.at[s],
                    send_sem=s2_sems.at[s],
                    recv_sem=r2_sems.at[s],
                    device_id=(0,),
                    device_id_type=pl.DeviceIdType.MESH,
                )
                recv.wait_recv()
            out_ref[s // 4, (s % 4) * CH2:((s % 4) + 1) * CH2, :] = \
                rbuf2_ref[s, :, :].astype(jnp.float32)

        for t in range(N_DEV):
            @pl.when(t != my)
            def _(t=t):
                w1 = pltpu.make_async_remote_copy(
                    src_ref=rbuf1_ref.at[t], dst_ref=rbuf1_ref.at[t],
                    send_sem=s1_sems.at[t], recv_sem=r1_sems.at[t],
                    device_id=(0,), device_id_type=pl.DeviceIdType.MESH,
                )
                w1.wait_send()
                w2 = pltpu.make_async_remote_copy(
                    src_ref=rbuf2_ref.at[t], dst_ref=rbuf2_ref.at[t],
                    send_sem=s2_sems.at[t], recv_sem=r2_sems.at[t],
                    device_id=(0,), device_id_type=pl.DeviceIdType.MESH,
                )
                w2.wait_send()

    A = jnp.concatenate([x.reshape(R, D), Wo], axis=0)
    Bm = jnp.concatenate([Wq, Wk, Wv,
                          jnp.asarray(cos_np), jnp.asarray(sin_np)],
                         axis=0)

    return pl.pallas_call(
        body,
        out_shape=jax.ShapeDtypeStruct((B, SQ, D), jnp.float32),
        in_specs=[pl.BlockSpec(memory_space=pltpu.VMEM)] * 2,
        out_specs=pl.BlockSpec(memory_space=pltpu.VMEM),
        scratch_shapes=[
            pltpu.VMEM((R, D), jnp.bfloat16),
            pltpu.VMEM((N_DEV, CH2, D), jnp.bfloat16),
            pltpu.VMEM((N_DEV, CH2, D), jnp.bfloat16),
            pltpu.SemaphoreType.DMA((N_DEV,)),
            pltpu.SemaphoreType.DMA((N_DEV,)),
            pltpu.SemaphoreType.DMA((N_DEV,)),
            pltpu.SemaphoreType.DMA((N_DEV,)),
        ],
        compiler_params=pltpu.CompilerParams(collective_id=0),
    )(A, Bm)
